# baseline (speedup 1.0000x reference)
import functools

import jax
import jax.numpy as jnp
from jax import lax
from jax.experimental import pallas as pl
from jax.experimental.pallas import tpu as pltpu
from jax.experimental.pallas import tpu_sc as plsc

N = 10000
E = 320000
B = 64
D = 128
DOUT = 64

NC, NS = 2, 16
NW = NC * NS
CH = 128
CPW = 80
EPAD = NW * CPW * CH
NROWS = EPAD // CH
NPAD = 10112
RPT = NPAD // NS

GRP = 8
NBUF = 2
CPW0 = 80
CPW1 = 80
NGRP0 = CPW0 // GRP
NGRP1 = CPW1 // GRP

_mesh = plsc.VectorSubcoreMesh(core_axis_name="c", subcore_axis_name="s")


def _sc_count_body(dst2, out, dst_v, hist, sem):
    c = lax.axis_index("c")
    s = lax.axis_index("s")
    wid = c * NS + s
    pltpu.sync_copy(dst2.at[pl.ds(wid * CPW, CPW)], dst_v)
    zeros16 = jnp.zeros((16,), jnp.float32)
    ones16 = jnp.ones((16,), jnp.float32)

    def zbody(i, carry):
        hist[pl.ds(i * 16, 16)] = zeros16
        return carry

    lax.fori_loop(0, NPAD // 16, zbody, 0)

    def body(i, carry):
        idx = dst_v[i // (CH // 16), pl.ds((i % (CH // 16)) * 16, 16)]
        plsc.addupdate_scatter(hist, [idx], ones16)
        return carry

    lax.fori_loop(0, (CPW * CH) // 16, body, 0)
    pltpu.sync_copy(hist, out.at[wid])


_sc_count = functools.partial(
    pl.kernel,
    _sc_count_body,
    out_type=jax.ShapeDtypeStruct((NW, NPAD), jnp.float32),
    mesh=_mesh,
    compiler_params=pltpu.CompilerParams(needs_layout_passes=False),
    scratch_types=[
        pltpu.VMEM((CPW, CH), jnp.int32),
        pltpu.VMEM((NPAD,), jnp.float32),
        pltpu.SemaphoreType.DMA,
    ],
)()


def _sc_agg_body(hp, src2, dst2, zeros, out, srcg, dstg, rows_v, acc,
                 gsem0, gsem1, ssem0, ssem1, isem):
    c = lax.axis_index("c")
    s = lax.axis_index("s")
    gsems = (gsem0, gsem1)
    ssems = (ssem0, ssem1)
    base = pl.multiple_of(
        jnp.where(c == 0, s * CPW0, NS * CPW0 + s * CPW1), 8)
    cpw = jnp.where(c == 0, CPW0, CPW1)
    ngrp = jnp.where(c == 0, NGRP0, NGRP1)
    pltpu.sync_copy(src2.at[pl.ds(base, GRP)], srcg.at[0])
    pltpu.sync_copy(dst2.at[pl.ds(base, GRP)], dstg.at[0])
    gbase1 = pl.multiple_of(base + GRP, 8)
    pltpu.async_copy(src2.at[pl.ds(gbase1, GRP)], srcg.at[1], isem)
    pltpu.async_copy(dst2.at[pl.ds(gbase1, GRP)], dstg.at[1], isem)
    pltpu.sync_copy(zeros, acc.at[pl.ds(s * RPT, RPT)])
    plsc.subcore_barrier()
    pltpu.async_copy(hp.at[srcg.at[0, 0]], rows_v.at[0], gsem0)

    def body(k, carry):
        g = k // GRP
        p = g % 2
        q = (g + 1) % 2
        r = k % GRP

        for b in range(NBUF):
            @pl.when(k % NBUF == b)
            def _():
                b2 = (b + 1) % NBUF
                pltpu.make_async_copy(hp.at[srcg.at[p, r]], rows_v.at[b],
                                      gsems[b]).wait()
                pltpu.async_copy(rows_v.at[b], acc.at[dstg.at[p, r]],
                                 ssems[b], add=True)

                @pl.when(k > 0)
                def _():
                    pltpu.make_async_copy(
                        rows_v.at[b2], acc.at[dstg.at[p, r]],
                        ssems[b2]).wait()

                @pl.when(jnp.logical_and(jnp.logical_and(r == 0, k > 0),
                                         g + 1 < ngrp))
                def _():
                    gb = pl.multiple_of(base + (g + 1) * GRP, 8)
                    pltpu.async_copy(src2.at[pl.ds(gb, GRP)], srcg.at[q],
                                     isem)
                    pltpu.async_copy(dst2.at[pl.ds(gb, GRP)], dstg.at[q],
                                     isem)

                @pl.when(jnp.logical_and(r == GRP - 1, g + 1 < ngrp))
                def _():
                    pltpu.make_async_copy(src2.at[pl.ds(0, GRP)], srcg.at[q],
                                          isem).wait()
                    pltpu.make_async_copy(dst2.at[pl.ds(0, GRP)], dstg.at[q],
                                          isem).wait()

                @pl.when(k + 1 < cpw)
                def _():
                    g2 = (k + 1) // GRP
                    r2 = (k + 1) % GRP
                    pltpu.async_copy(hp.at[srcg.at[g2 % 2, r2]],
                                     rows_v.at[b2], gsems[b2])
        return carry

    lax.fori_loop(0, cpw, body, 0)
    for b in range(NBUF):
        @pl.when((cpw - 1) % NBUF == b)
        def _():
            pltpu.make_async_copy(rows_v.at[b], acc.at[dstg.at[0, 0]],
                                  ssems[b]).wait()
    plsc.subcore_barrier()
    pltpu.sync_copy(acc.at[pl.ds(s * RPT, RPT)], out.at[c, pl.ds(s * RPT, RPT)])


_sc_agg = functools.partial(
    pl.kernel,
    _sc_agg_body,
    out_type=jax.ShapeDtypeStruct((NC, NPAD, D), jnp.float32),
    mesh=_mesh,
    scratch_types=[
        pltpu.VMEM((2, GRP, CH), jnp.int32),
        pltpu.VMEM((2, GRP, CH), jnp.int32),
        pltpu.VMEM((NBUF, CH, D), jnp.float32),
        pltpu.VMEM_SHARED((NPAD, D), jnp.float32),
        pltpu.SemaphoreType.DMA,
        pltpu.SemaphoreType.DMA,
        pltpu.SemaphoreType.DMA,
        pltpu.SemaphoreType.DMA,
        pltpu.SemaphoreType.DMA,
    ],
)()


def _tc_prep_body(x_ref, w1_ref, cp_ref, z_ref, hp_ref, dinv_ref):
    cnt = jnp.sum(cp_ref[...], axis=0)[:N].reshape(N, 1)
    dinv = lax.rsqrt(cnt + 1.0)
    z = jnp.dot(x_ref[...], w1_ref[...], preferred_element_type=jnp.float32)
    dinv_ref[...] = dinv
    z_ref[...] = z
    hp_ref[...] = z * dinv


def _tc_mid_body(s1_ref, z1_ref, dinv_ref, b1_ref, w2_ref, z2_ref, h2p_ref):
    dinv = dinv_ref[...]
    s = s1_ref[0, :N, :] + s1_ref[1, :N, :]
    z1 = z1_ref[...]
    h1 = jnp.maximum(dinv * s + dinv * dinv * z1 + b1_ref[...], 0.0)
    z2 = jnp.dot(h1, w2_ref[...], preferred_element_type=jnp.float32)
    z2_ref[...] = z2
    h2p_ref[...] = z2 * dinv


def _tc_fin_body(s2_ref, z2_ref, dinv_ref, b2_ref, bat_ref, wfc_ref, bfc_ref,
                 out_ref):
    dinv = dinv_ref[...]
    s = s2_ref[0, :N, :] + s2_ref[1, :N, :]
    z2 = z2_ref[...]
    h2 = jnp.maximum(dinv * s + dinv * dinv * z2 + b2_ref[...], 0.0)
    ids = lax.broadcasted_iota(jnp.int32, (B, N), 0)
    m = (bat_ref[...] == ids).astype(jnp.float32)
    sums = jnp.dot(m, h2, preferred_element_type=jnp.float32)
    cb = jnp.sum(m, axis=1, keepdims=True)
    pooled = sums / jnp.maximum(cb, 1.0)
    logits = jnp.dot(pooled, wfc_ref[...],
                     preferred_element_type=jnp.float32) + bfc_ref[...]
    mx = jnp.max(logits, axis=1, keepdims=True)
    sh = logits - mx
    out_ref[...] = sh - jnp.log(jnp.sum(jnp.exp(sh), axis=1, keepdims=True))


_tc_prep = pl.pallas_call(
    _tc_prep_body,
    out_shape=[
        jax.ShapeDtypeStruct((N, D), jnp.float32),
        jax.ShapeDtypeStruct((N, D), jnp.float32),
        jax.ShapeDtypeStruct((N, 1), jnp.float32),
    ],
)

_tc_mid = pl.pallas_call(
    _tc_mid_body,
    out_shape=[
        jax.ShapeDtypeStruct((N, D), jnp.float32),
        jax.ShapeDtypeStruct((N, D), jnp.float32),
    ],
)

_tc_fin = pl.pallas_call(
    _tc_fin_body,
    out_shape=jax.ShapeDtypeStruct((B, DOUT), jnp.float32),
)


def kernel(x, edge_index, batch, W1, b1, W2, b2, Wfc, bfc):
    src = edge_index[0]
    dst = edge_index[1]
    pad = EPAD - E
    src2 = jnp.concatenate([src, jnp.zeros((pad,), jnp.int32)]).reshape(NROWS, CH)
    dst2 = jnp.concatenate([dst, jnp.full((pad,), N, jnp.int32)]).reshape(NROWS, CH)
    zeros_d = jnp.zeros((RPT, D), jnp.float32)

    cp = _sc_count(dst2)
    z1, h1p, dinv = _tc_prep(x, W1, cp)
    s1 = _sc_agg(h1p, src2, dst2, zeros_d)
    z2, h2p = _tc_mid(s1, z1, dinv, b1.reshape(1, D), W2)
    s2 = _sc_agg(h2p, src2, dst2, zeros_d)
    return _tc_fin(s2, z2, dinv, b2.reshape(1, D), batch.reshape(1, N),
                   Wfc, bfc.reshape(1, DOUT))


# speedup vs baseline: 1.1135x; 1.1135x over previous
import functools

import jax
import jax.numpy as jnp
from jax import lax
from jax.experimental import pallas as pl
from jax.experimental.pallas import tpu as pltpu
from jax.experimental.pallas import tpu_sc as plsc

N = 10000
E = 320000
B = 64
D = 128
DOUT = 64

NC, NS = 2, 16
NW = NC * NS
CH = 128
CPW = 80
EPAD = NW * CPW * CH
NROWS = EPAD // CH
NPAD = 10112
RPT = NPAD // NS

GRP = 8
NBUF = 2
CPW0 = 120
CPW1 = 40
NGRP0 = CPW0 // GRP
NGRP1 = CPW1 // GRP

_mesh = plsc.VectorSubcoreMesh(core_axis_name="c", subcore_axis_name="s")


def _sc_count_body(dst2, out, dst_v, hist, sem):
    c = lax.axis_index("c")
    s = lax.axis_index("s")
    wid = c * NS + s
    pltpu.sync_copy(dst2.at[pl.ds(wid * CPW, CPW)], dst_v)
    zeros16 = jnp.zeros((16,), jnp.float32)
    ones16 = jnp.ones((16,), jnp.float32)

    def zbody(i, carry):
        hist[pl.ds(i * 16, 16)] = zeros16
        return carry

    lax.fori_loop(0, NPAD // 16, zbody, 0)

    def body(i, carry):
        idx = dst_v[i // (CH // 16), pl.ds((i % (CH // 16)) * 16, 16)]
        plsc.addupdate_scatter(hist, [idx], ones16)
        return carry

    lax.fori_loop(0, (CPW * CH) // 16, body, 0)
    pltpu.sync_copy(hist, out.at[wid])


_sc_count = functools.partial(
    pl.kernel,
    _sc_count_body,
    out_type=jax.ShapeDtypeStruct((NW, NPAD), jnp.float32),
    mesh=_mesh,
    compiler_params=pltpu.CompilerParams(needs_layout_passes=False),
    scratch_types=[
        pltpu.VMEM((CPW, CH), jnp.int32),
        pltpu.VMEM((NPAD,), jnp.float32),
        pltpu.SemaphoreType.DMA,
    ],
)()


def _sc_agg_body(hp, src2, dst2, zeros, out, srcg, dstg, rows_v, acc,
                 gsem0, gsem1, ssem0, ssem1, isem):
    c = lax.axis_index("c")
    s = lax.axis_index("s")
    gsems = (gsem0, gsem1)
    ssems = (ssem0, ssem1)
    base = pl.multiple_of(
        jnp.where(c == 0, s * CPW0, NS * CPW0 + s * CPW1), 8)
    cpw = jnp.where(c == 0, CPW0, CPW1)
    ngrp = jnp.where(c == 0, NGRP0, NGRP1)
    pltpu.sync_copy(src2.at[pl.ds(base, GRP)], srcg.at[0])
    pltpu.sync_copy(dst2.at[pl.ds(base, GRP)], dstg.at[0])
    gbase1 = pl.multiple_of(base + GRP, 8)
    pltpu.async_copy(src2.at[pl.ds(gbase1, GRP)], srcg.at[1], isem)
    pltpu.async_copy(dst2.at[pl.ds(gbase1, GRP)], dstg.at[1], isem)
    pltpu.sync_copy(zeros, acc.at[pl.ds(s * RPT, RPT)])
    plsc.subcore_barrier()
    pltpu.async_copy(hp.at[srcg.at[0, 0]], rows_v.at[0], gsem0)

    def body(k, carry):
        g = k // GRP
        p = g % 2
        q = (g + 1) % 2
        r = k % GRP

        for b in range(NBUF):
            @pl.when(k % NBUF == b)
            def _():
                b2 = (b + 1) % NBUF
                pltpu.make_async_copy(hp.at[srcg.at[p, r]], rows_v.at[b],
                                      gsems[b]).wait()
                pltpu.async_copy(rows_v.at[b], acc.at[dstg.at[p, r]],
                                 ssems[b], add=True)

                @pl.when(k > 0)
                def _():
                    pltpu.make_async_copy(
                        rows_v.at[b2], acc.at[dstg.at[p, r]],
                        ssems[b2]).wait()

                @pl.when(jnp.logical_and(jnp.logical_and(r == 0, k > 0),
                                         g + 1 < ngrp))
                def _():
                    gb = pl.multiple_of(base + (g + 1) * GRP, 8)
                    pltpu.async_copy(src2.at[pl.ds(gb, GRP)], srcg.at[q],
                                     isem)
                    pltpu.async_copy(dst2.at[pl.ds(gb, GRP)], dstg.at[q],
                                     isem)

                @pl.when(jnp.logical_and(r == GRP - 1, g + 1 < ngrp))
                def _():
                    pltpu.make_async_copy(src2.at[pl.ds(0, GRP)], srcg.at[q],
                                          isem).wait()
                    pltpu.make_async_copy(dst2.at[pl.ds(0, GRP)], dstg.at[q],
                                          isem).wait()

                @pl.when(k + 1 < cpw)
                def _():
                    g2 = (k + 1) // GRP
                    r2 = (k + 1) % GRP
                    pltpu.async_copy(hp.at[srcg.at[g2 % 2, r2]],
                                     rows_v.at[b2], gsems[b2])
        return carry

    lax.fori_loop(0, cpw, body, 0)
    for b in range(NBUF):
        @pl.when((cpw - 1) % NBUF == b)
        def _():
            pltpu.make_async_copy(rows_v.at[b], acc.at[dstg.at[0, 0]],
                                  ssems[b]).wait()
    plsc.subcore_barrier()
    pltpu.sync_copy(acc.at[pl.ds(s * RPT, RPT)], out.at[c, pl.ds(s * RPT, RPT)])


_sc_agg = functools.partial(
    pl.kernel,
    _sc_agg_body,
    out_type=jax.ShapeDtypeStruct((NC, NPAD, D), jnp.float32),
    mesh=_mesh,
    scratch_types=[
        pltpu.VMEM((2, GRP, CH), jnp.int32),
        pltpu.VMEM((2, GRP, CH), jnp.int32),
        pltpu.VMEM((NBUF, CH, D), jnp.float32),
        pltpu.VMEM_SHARED((NPAD, D), jnp.float32),
        pltpu.SemaphoreType.DMA,
        pltpu.SemaphoreType.DMA,
        pltpu.SemaphoreType.DMA,
        pltpu.SemaphoreType.DMA,
        pltpu.SemaphoreType.DMA,
    ],
)()


def _tc_prep_body(x_ref, w1_ref, cp_ref, z_ref, hp_ref, dinv_ref):
    cnt = jnp.sum(cp_ref[...], axis=0)[:N].reshape(N, 1)
    dinv = lax.rsqrt(cnt + 1.0)
    z = jnp.dot(x_ref[...], w1_ref[...], preferred_element_type=jnp.float32)
    dinv_ref[...] = dinv
    z_ref[...] = z
    hp_ref[...] = z * dinv


def _tc_mid_body(s1_ref, z1_ref, dinv_ref, b1_ref, w2_ref, z2_ref, h2p_ref):
    dinv = dinv_ref[...]
    s = s1_ref[0, :N, :] + s1_ref[1, :N, :]
    z1 = z1_ref[...]
    h1 = jnp.maximum(dinv * s + dinv * dinv * z1 + b1_ref[...], 0.0)
    z2 = jnp.dot(h1, w2_ref[...], preferred_element_type=jnp.float32)
    z2_ref[...] = z2
    h2p_ref[...] = z2 * dinv


def _tc_fin_body(s2_ref, z2_ref, dinv_ref, b2_ref, bat_ref, wfc_ref, bfc_ref,
                 out_ref):
    dinv = dinv_ref[...]
    s = s2_ref[0, :N, :] + s2_ref[1, :N, :]
    z2 = z2_ref[...]
    h2 = jnp.maximum(dinv * s + dinv * dinv * z2 + b2_ref[...], 0.0)
    ids = lax.broadcasted_iota(jnp.int32, (B, N), 0)
    m = (bat_ref[...] == ids).astype(jnp.float32)
    sums = jnp.dot(m, h2, preferred_element_type=jnp.float32)
    cb = jnp.sum(m, axis=1, keepdims=True)
    pooled = sums / jnp.maximum(cb, 1.0)
    logits = jnp.dot(pooled, wfc_ref[...],
                     preferred_element_type=jnp.float32) + bfc_ref[...]
    mx = jnp.max(logits, axis=1, keepdims=True)
    sh = logits - mx
    out_ref[...] = sh - jnp.log(jnp.sum(jnp.exp(sh), axis=1, keepdims=True))


_tc_prep = pl.pallas_call(
    _tc_prep_body,
    out_shape=[
        jax.ShapeDtypeStruct((N, D), jnp.float32),
        jax.ShapeDtypeStruct((N, D), jnp.float32),
        jax.ShapeDtypeStruct((N, 1), jnp.float32),
    ],
)

_tc_mid = pl.pallas_call(
    _tc_mid_body,
    out_shape=[
        jax.ShapeDtypeStruct((N, D), jnp.float32),
        jax.ShapeDtypeStruct((N, D), jnp.float32),
    ],
)

_tc_fin = pl.pallas_call(
    _tc_fin_body,
    out_shape=jax.ShapeDtypeStruct((B, DOUT), jnp.float32),
)


def kernel(x, edge_index, batch, W1, b1, W2, b2, Wfc, bfc):
    src = edge_index[0]
    dst = edge_index[1]
    pad = EPAD - E
    src2 = jnp.concatenate([src, jnp.zeros((pad,), jnp.int32)]).reshape(NROWS, CH)
    dst2 = jnp.concatenate([dst, jnp.full((pad,), N, jnp.int32)]).reshape(NROWS, CH)
    zeros_d = jnp.zeros((RPT, D), jnp.float32)

    cp = _sc_count(dst2)
    z1, h1p, dinv = _tc_prep(x, W1, cp)
    s1 = _sc_agg(h1p, src2, dst2, zeros_d)
    z2, h2p = _tc_mid(s1, z1, dinv, b1.reshape(1, D), W2)
    s2 = _sc_agg(h2p, src2, dst2, zeros_d)
    return _tc_fin(s2, z2, dinv, b2.reshape(1, D), batch.reshape(1, N),
                   Wfc, bfc.reshape(1, DOUT))


# speedup vs baseline: 1.1307x; 1.0154x over previous
import functools

import jax
import jax.numpy as jnp
from jax import lax
from jax.experimental import pallas as pl
from jax.experimental.pallas import tpu as pltpu
from jax.experimental.pallas import tpu_sc as plsc

N = 10000
E = 320000
B = 64
D = 128
DOUT = 64

NC, NS = 2, 16
NW = NC * NS
CH = 128
CPW = 80
EPAD = NW * CPW * CH
NROWS = EPAD // CH
NPAD = 10112
RPT = NPAD // NS

GRP = 8
NBUF = 2
CPW0 = 128
CPW1 = 32
NGRP0 = CPW0 // GRP
NGRP1 = CPW1 // GRP

_mesh = plsc.VectorSubcoreMesh(core_axis_name="c", subcore_axis_name="s")


def _sc_count_body(dst2, out, dst_v, hist, sem):
    c = lax.axis_index("c")
    s = lax.axis_index("s")
    wid = c * NS + s
    pltpu.sync_copy(dst2.at[pl.ds(wid * CPW, CPW)], dst_v)
    zeros16 = jnp.zeros((16,), jnp.float32)
    ones16 = jnp.ones((16,), jnp.float32)

    def zbody(i, carry):
        hist[pl.ds(i * 16, 16)] = zeros16
        return carry

    lax.fori_loop(0, NPAD // 16, zbody, 0)

    def body(i, carry):
        idx = dst_v[i // (CH // 16), pl.ds((i % (CH // 16)) * 16, 16)]
        plsc.addupdate_scatter(hist, [idx], ones16)
        return carry

    lax.fori_loop(0, (CPW * CH) // 16, body, 0)
    pltpu.sync_copy(hist, out.at[wid])


_sc_count = functools.partial(
    pl.kernel,
    _sc_count_body,
    out_type=jax.ShapeDtypeStruct((NW, NPAD), jnp.float32),
    mesh=_mesh,
    compiler_params=pltpu.CompilerParams(needs_layout_passes=False),
    scratch_types=[
        pltpu.VMEM((CPW, CH), jnp.int32),
        pltpu.VMEM((NPAD,), jnp.float32),
        pltpu.SemaphoreType.DMA,
    ],
)()


def _sc_agg_body(hp, src2, dst2, zeros, out, srcg, dstg, rows_v, acc,
                 gsem0, gsem1, ssem0, ssem1, isem):
    c = lax.axis_index("c")
    s = lax.axis_index("s")
    gsems = (gsem0, gsem1)
    ssems = (ssem0, ssem1)
    base = pl.multiple_of(
        jnp.where(c == 0, s * CPW0, NS * CPW0 + s * CPW1), 8)
    cpw = jnp.where(c == 0, CPW0, CPW1)
    ngrp = jnp.where(c == 0, NGRP0, NGRP1)
    pltpu.sync_copy(src2.at[pl.ds(base, GRP)], srcg.at[0])
    pltpu.sync_copy(dst2.at[pl.ds(base, GRP)], dstg.at[0])
    gbase1 = pl.multiple_of(base + GRP, 8)
    pltpu.async_copy(src2.at[pl.ds(gbase1, GRP)], srcg.at[1], isem)
    pltpu.async_copy(dst2.at[pl.ds(gbase1, GRP)], dstg.at[1], isem)
    pltpu.sync_copy(zeros, acc.at[pl.ds(s * RPT, RPT)])
    plsc.subcore_barrier()
    pltpu.async_copy(hp.at[srcg.at[0, 0]], rows_v.at[0], gsem0)

    def body(k, carry):
        g = k // GRP
        p = g % 2
        q = (g + 1) % 2
        r = k % GRP

        for b in range(NBUF):
            @pl.when(k % NBUF == b)
            def _():
                b2 = (b + 1) % NBUF
                pltpu.make_async_copy(hp.at[srcg.at[p, r]], rows_v.at[b],
                                      gsems[b]).wait()
                pltpu.async_copy(rows_v.at[b], acc.at[dstg.at[p, r]],
                                 ssems[b], add=True)

                @pl.when(k > 0)
                def _():
                    pltpu.make_async_copy(
                        rows_v.at[b2], acc.at[dstg.at[p, r]],
                        ssems[b2]).wait()

                @pl.when(jnp.logical_and(jnp.logical_and(r == 0, k > 0),
                                         g + 1 < ngrp))
                def _():
                    gb = pl.multiple_of(base + (g + 1) * GRP, 8)
                    pltpu.async_copy(src2.at[pl.ds(gb, GRP)], srcg.at[q],
                                     isem)
                    pltpu.async_copy(dst2.at[pl.ds(gb, GRP)], dstg.at[q],
                                     isem)

                @pl.when(jnp.logical_and(r == GRP - 1, g + 1 < ngrp))
                def _():
                    pltpu.make_async_copy(src2.at[pl.ds(0, GRP)], srcg.at[q],
                                          isem).wait()
                    pltpu.make_async_copy(dst2.at[pl.ds(0, GRP)], dstg.at[q],
                                          isem).wait()

                @pl.when(k + 1 < cpw)
                def _():
                    g2 = (k + 1) // GRP
                    r2 = (k + 1) % GRP
                    pltpu.async_copy(hp.at[srcg.at[g2 % 2, r2]],
                                     rows_v.at[b2], gsems[b2])
        return carry

    lax.fori_loop(0, cpw, body, 0)
    for b in range(NBUF):
        @pl.when((cpw - 1) % NBUF == b)
        def _():
            pltpu.make_async_copy(rows_v.at[b], acc.at[dstg.at[0, 0]],
                                  ssems[b]).wait()
    plsc.subcore_barrier()
    pltpu.sync_copy(acc.at[pl.ds(s * RPT, RPT)], out.at[c, pl.ds(s * RPT, RPT)])


_sc_agg = functools.partial(
    pl.kernel,
    _sc_agg_body,
    out_type=jax.ShapeDtypeStruct((NC, NPAD, D), jnp.float32),
    mesh=_mesh,
    scratch_types=[
        pltpu.VMEM((2, GRP, CH), jnp.int32),
        pltpu.VMEM((2, GRP, CH), jnp.int32),
        pltpu.VMEM((NBUF, CH, D), jnp.float32),
        pltpu.VMEM_SHARED((NPAD, D), jnp.float32),
        pltpu.SemaphoreType.DMA,
        pltpu.SemaphoreType.DMA,
        pltpu.SemaphoreType.DMA,
        pltpu.SemaphoreType.DMA,
        pltpu.SemaphoreType.DMA,
    ],
)()


def _tc_prep_body(x_ref, w1_ref, cp_ref, z_ref, hp_ref, dinv_ref):
    cnt = jnp.sum(cp_ref[...], axis=0)[:N].reshape(N, 1)
    dinv = lax.rsqrt(cnt + 1.0)
    z = jnp.dot(x_ref[...], w1_ref[...], preferred_element_type=jnp.float32)
    dinv_ref[...] = dinv
    z_ref[...] = z
    hp_ref[...] = z * dinv


def _tc_mid_body(s1_ref, z1_ref, dinv_ref, b1_ref, w2_ref, z2_ref, h2p_ref):
    dinv = dinv_ref[...]
    s = s1_ref[0, :N, :] + s1_ref[1, :N, :]
    z1 = z1_ref[...]
    h1 = jnp.maximum(dinv * s + dinv * dinv * z1 + b1_ref[...], 0.0)
    z2 = jnp.dot(h1, w2_ref[...], preferred_element_type=jnp.float32)
    z2_ref[...] = z2
    h2p_ref[...] = z2 * dinv


def _tc_fin_body(s2_ref, z2_ref, dinv_ref, b2_ref, bat_ref, wfc_ref, bfc_ref,
                 out_ref):
    dinv = dinv_ref[...]
    s = s2_ref[0, :N, :] + s2_ref[1, :N, :]
    z2 = z2_ref[...]
    h2 = jnp.maximum(dinv * s + dinv * dinv * z2 + b2_ref[...], 0.0)
    ids = lax.broadcasted_iota(jnp.int32, (B, N), 0)
    m = (bat_ref[...] == ids).astype(jnp.float32)
    sums = jnp.dot(m, h2, preferred_element_type=jnp.float32)
    cb = jnp.sum(m, axis=1, keepdims=True)
    pooled = sums / jnp.maximum(cb, 1.0)
    logits = jnp.dot(pooled, wfc_ref[...],
                     preferred_element_type=jnp.float32) + bfc_ref[...]
    mx = jnp.max(logits, axis=1, keepdims=True)
    sh = logits - mx
    out_ref[...] = sh - jnp.log(jnp.sum(jnp.exp(sh), axis=1, keepdims=True))


_tc_prep = pl.pallas_call(
    _tc_prep_body,
    out_shape=[
        jax.ShapeDtypeStruct((N, D), jnp.float32),
        jax.ShapeDtypeStruct((N, D), jnp.float32),
        jax.ShapeDtypeStruct((N, 1), jnp.float32),
    ],
)

_tc_mid = pl.pallas_call(
    _tc_mid_body,
    out_shape=[
        jax.ShapeDtypeStruct((N, D), jnp.float32),
        jax.ShapeDtypeStruct((N, D), jnp.float32),
    ],
)

_tc_fin = pl.pallas_call(
    _tc_fin_body,
    out_shape=jax.ShapeDtypeStruct((B, DOUT), jnp.float32),
)


def kernel(x, edge_index, batch, W1, b1, W2, b2, Wfc, bfc):
    src = edge_index[0]
    dst = edge_index[1]
    pad = EPAD - E
    src2 = jnp.concatenate([src, jnp.zeros((pad,), jnp.int32)]).reshape(NROWS, CH)
    dst2 = jnp.concatenate([dst, jnp.full((pad,), N, jnp.int32)]).reshape(NROWS, CH)
    zeros_d = jnp.zeros((RPT, D), jnp.float32)

    cp = _sc_count(dst2)
    z1, h1p, dinv = _tc_prep(x, W1, cp)
    s1 = _sc_agg(h1p, src2, dst2, zeros_d)
    z2, h2p = _tc_mid(s1, z1, dinv, b1.reshape(1, D), W2)
    s2 = _sc_agg(h2p, src2, dst2, zeros_d)
    return _tc_fin(s2, z2, dinv, b2.reshape(1, D), batch.reshape(1, N),
                   Wfc, bfc.reshape(1, DOUT))


# speedup vs baseline: 1.1695x; 1.0343x over previous
import functools

import jax
import jax.numpy as jnp
from jax import lax
from jax.experimental import pallas as pl
from jax.experimental.pallas import tpu as pltpu
from jax.experimental.pallas import tpu_sc as plsc

N = 10000
E = 320000
B = 64
D = 128
DOUT = 64

NC, NS = 2, 16
NW = NC * NS
CH = 128
CPW = 80
EPAD = NW * CPW * CH
NROWS = EPAD // CH
NPAD = 10112
RPT = NPAD // NS

GRP = 8
NBUF = 2
CPW0 = 136
CPW1 = 24
NGRP0 = CPW0 // GRP
NGRP1 = CPW1 // GRP

_mesh = plsc.VectorSubcoreMesh(core_axis_name="c", subcore_axis_name="s")


def _sc_count_body(dst2, out, dst_v, hist, sem):
    c = lax.axis_index("c")
    s = lax.axis_index("s")
    wid = c * NS + s
    pltpu.sync_copy(dst2.at[pl.ds(wid * CPW, CPW)], dst_v)
    zeros16 = jnp.zeros((16,), jnp.float32)
    ones16 = jnp.ones((16,), jnp.float32)

    def zbody(i, carry):
        hist[pl.ds(i * 16, 16)] = zeros16
        return carry

    lax.fori_loop(0, NPAD // 16, zbody, 0)

    def body(i, carry):
        idx = dst_v[i // (CH // 16), pl.ds((i % (CH // 16)) * 16, 16)]
        plsc.addupdate_scatter(hist, [idx], ones16)
        return carry

    lax.fori_loop(0, (CPW * CH) // 16, body, 0)
    pltpu.sync_copy(hist, out.at[wid])


_sc_count = functools.partial(
    pl.kernel,
    _sc_count_body,
    out_type=jax.ShapeDtypeStruct((NW, NPAD), jnp.float32),
    mesh=_mesh,
    compiler_params=pltpu.CompilerParams(needs_layout_passes=False),
    scratch_types=[
        pltpu.VMEM((CPW, CH), jnp.int32),
        pltpu.VMEM((NPAD,), jnp.float32),
        pltpu.SemaphoreType.DMA,
    ],
)()


def _sc_agg_body(hp, src2, dst2, zeros, out, srcg, dstg, rows_v, acc,
                 gsem0, gsem1, ssem0, ssem1, isem):
    c = lax.axis_index("c")
    s = lax.axis_index("s")
    gsems = (gsem0, gsem1)
    ssems = (ssem0, ssem1)
    base = pl.multiple_of(
        jnp.where(c == 0, s * CPW0, NS * CPW0 + s * CPW1), 8)
    cpw = jnp.where(c == 0, CPW0, CPW1)
    ngrp = jnp.where(c == 0, NGRP0, NGRP1)
    pltpu.sync_copy(src2.at[pl.ds(base, GRP)], srcg.at[0])
    pltpu.sync_copy(dst2.at[pl.ds(base, GRP)], dstg.at[0])
    gbase1 = pl.multiple_of(base + GRP, 8)
    pltpu.async_copy(src2.at[pl.ds(gbase1, GRP)], srcg.at[1], isem)
    pltpu.async_copy(dst2.at[pl.ds(gbase1, GRP)], dstg.at[1], isem)
    pltpu.sync_copy(zeros, acc.at[pl.ds(s * RPT, RPT)])
    plsc.subcore_barrier()
    pltpu.async_copy(hp.at[srcg.at[0, 0]], rows_v.at[0], gsem0)

    def body(k, carry):
        g = k // GRP
        p = g % 2
        q = (g + 1) % 2
        r = k % GRP

        for b in range(NBUF):
            @pl.when(k % NBUF == b)
            def _():
                b2 = (b + 1) % NBUF
                pltpu.make_async_copy(hp.at[srcg.at[p, r]], rows_v.at[b],
                                      gsems[b]).wait()
                pltpu.async_copy(rows_v.at[b], acc.at[dstg.at[p, r]],
                                 ssems[b], add=True)

                @pl.when(k > 0)
                def _():
                    pltpu.make_async_copy(
                        rows_v.at[b2], acc.at[dstg.at[p, r]],
                        ssems[b2]).wait()

                @pl.when(jnp.logical_and(jnp.logical_and(r == 0, k > 0),
                                         g + 1 < ngrp))
                def _():
                    gb = pl.multiple_of(base + (g + 1) * GRP, 8)
                    pltpu.async_copy(src2.at[pl.ds(gb, GRP)], srcg.at[q],
                                     isem)
                    pltpu.async_copy(dst2.at[pl.ds(gb, GRP)], dstg.at[q],
                                     isem)

                @pl.when(jnp.logical_and(r == GRP - 1, g + 1 < ngrp))
                def _():
                    pltpu.make_async_copy(src2.at[pl.ds(0, GRP)], srcg.at[q],
                                          isem).wait()
                    pltpu.make_async_copy(dst2.at[pl.ds(0, GRP)], dstg.at[q],
                                          isem).wait()

                @pl.when(k + 1 < cpw)
                def _():
                    g2 = (k + 1) // GRP
                    r2 = (k + 1) % GRP
                    pltpu.async_copy(hp.at[srcg.at[g2 % 2, r2]],
                                     rows_v.at[b2], gsems[b2])
        return carry

    lax.fori_loop(0, cpw, body, 0)
    for b in range(NBUF):
        @pl.when((cpw - 1) % NBUF == b)
        def _():
            pltpu.make_async_copy(rows_v.at[b], acc.at[dstg.at[0, 0]],
                                  ssems[b]).wait()
    plsc.subcore_barrier()
    pltpu.sync_copy(acc.at[pl.ds(s * RPT, RPT)], out.at[c, pl.ds(s * RPT, RPT)])


_sc_agg = functools.partial(
    pl.kernel,
    _sc_agg_body,
    out_type=jax.ShapeDtypeStruct((NC, NPAD, D), jnp.float32),
    mesh=_mesh,
    scratch_types=[
        pltpu.VMEM((2, GRP, CH), jnp.int32),
        pltpu.VMEM((2, GRP, CH), jnp.int32),
        pltpu.VMEM((NBUF, CH, D), jnp.float32),
        pltpu.VMEM_SHARED((NPAD, D), jnp.float32),
        pltpu.SemaphoreType.DMA,
        pltpu.SemaphoreType.DMA,
        pltpu.SemaphoreType.DMA,
        pltpu.SemaphoreType.DMA,
        pltpu.SemaphoreType.DMA,
    ],
)()


def _tc_prep_body(x_ref, w1_ref, cp_ref, z_ref, hp_ref, dinv_ref):
    cnt = jnp.sum(cp_ref[...], axis=0)[:N].reshape(N, 1)
    dinv = lax.rsqrt(cnt + 1.0)
    z = jnp.dot(x_ref[...], w1_ref[...], preferred_element_type=jnp.float32)
    dinv_ref[...] = dinv
    z_ref[...] = z
    hp_ref[...] = z * dinv


def _tc_mid_body(s1_ref, z1_ref, dinv_ref, b1_ref, w2_ref, z2_ref, h2p_ref):
    dinv = dinv_ref[...]
    s = s1_ref[0, :N, :] + s1_ref[1, :N, :]
    z1 = z1_ref[...]
    h1 = jnp.maximum(dinv * s + dinv * dinv * z1 + b1_ref[...], 0.0)
    z2 = jnp.dot(h1, w2_ref[...], preferred_element_type=jnp.float32)
    z2_ref[...] = z2
    h2p_ref[...] = z2 * dinv


def _tc_fin_body(s2_ref, z2_ref, dinv_ref, b2_ref, bat_ref, wfc_ref, bfc_ref,
                 out_ref):
    dinv = dinv_ref[...]
    s = s2_ref[0, :N, :] + s2_ref[1, :N, :]
    z2 = z2_ref[...]
    h2 = jnp.maximum(dinv * s + dinv * dinv * z2 + b2_ref[...], 0.0)
    ids = lax.broadcasted_iota(jnp.int32, (B, N), 0)
    m = (bat_ref[...] == ids).astype(jnp.float32)
    sums = jnp.dot(m, h2, preferred_element_type=jnp.float32)
    cb = jnp.sum(m, axis=1, keepdims=True)
    pooled = sums / jnp.maximum(cb, 1.0)
    logits = jnp.dot(pooled, wfc_ref[...],
                     preferred_element_type=jnp.float32) + bfc_ref[...]
    mx = jnp.max(logits, axis=1, keepdims=True)
    sh = logits - mx
    out_ref[...] = sh - jnp.log(jnp.sum(jnp.exp(sh), axis=1, keepdims=True))


_tc_prep = pl.pallas_call(
    _tc_prep_body,
    out_shape=[
        jax.ShapeDtypeStruct((N, D), jnp.float32),
        jax.ShapeDtypeStruct((N, D), jnp.float32),
        jax.ShapeDtypeStruct((N, 1), jnp.float32),
    ],
)

_tc_mid = pl.pallas_call(
    _tc_mid_body,
    out_shape=[
        jax.ShapeDtypeStruct((N, D), jnp.float32),
        jax.ShapeDtypeStruct((N, D), jnp.float32),
    ],
)

_tc_fin = pl.pallas_call(
    _tc_fin_body,
    out_shape=jax.ShapeDtypeStruct((B, DOUT), jnp.float32),
)


def kernel(x, edge_index, batch, W1, b1, W2, b2, Wfc, bfc):
    src = edge_index[0]
    dst = edge_index[1]
    pad = EPAD - E
    src2 = jnp.concatenate([src, jnp.zeros((pad,), jnp.int32)]).reshape(NROWS, CH)
    dst2 = jnp.concatenate([dst, jnp.full((pad,), N, jnp.int32)]).reshape(NROWS, CH)
    zeros_d = jnp.zeros((RPT, D), jnp.float32)

    cp = _sc_count(dst2)
    z1, h1p, dinv = _tc_prep(x, W1, cp)
    s1 = _sc_agg(h1p, src2, dst2, zeros_d)
    z2, h2p = _tc_mid(s1, z1, dinv, b1.reshape(1, D), W2)
    s2 = _sc_agg(h2p, src2, dst2, zeros_d)
    return _tc_fin(s2, z2, dinv, b2.reshape(1, D), batch.reshape(1, N),
                   Wfc, bfc.reshape(1, DOUT))


# speedup vs baseline: 1.2639x; 1.0807x over previous
import functools

import jax
import jax.numpy as jnp
from jax import lax
from jax.experimental import pallas as pl
from jax.experimental.pallas import tpu as pltpu
from jax.experimental.pallas import tpu_sc as plsc

N = 10000
E = 320000
B = 64
D = 128
DOUT = 64

NC, NS = 2, 16
NW = NC * NS
CH = 128
CPW = 80
EPAD = NW * CPW * CH
NROWS = EPAD // CH
NPAD = 10112
RPT = NPAD // NS

GRP = 8
NBUF = 2
CPW0 = 144
CPW1 = 16
NGRP0 = CPW0 // GRP
NGRP1 = CPW1 // GRP

_mesh = plsc.VectorSubcoreMesh(core_axis_name="c", subcore_axis_name="s")


def _sc_count_body(dst2, out, dst_v, hist, sem):
    c = lax.axis_index("c")
    s = lax.axis_index("s")
    wid = c * NS + s
    pltpu.sync_copy(dst2.at[pl.ds(wid * CPW, CPW)], dst_v)
    zeros16 = jnp.zeros((16,), jnp.float32)
    ones16 = jnp.ones((16,), jnp.float32)

    def zbody(i, carry):
        hist[pl.ds(i * 16, 16)] = zeros16
        return carry

    lax.fori_loop(0, NPAD // 16, zbody, 0)

    def body(i, carry):
        idx = dst_v[i // (CH // 16), pl.ds((i % (CH // 16)) * 16, 16)]
        plsc.addupdate_scatter(hist, [idx], ones16)
        return carry

    lax.fori_loop(0, (CPW * CH) // 16, body, 0)
    pltpu.sync_copy(hist, out.at[wid])


_sc_count = functools.partial(
    pl.kernel,
    _sc_count_body,
    out_type=jax.ShapeDtypeStruct((NW, NPAD), jnp.float32),
    mesh=_mesh,
    compiler_params=pltpu.CompilerParams(needs_layout_passes=False),
    scratch_types=[
        pltpu.VMEM((CPW, CH), jnp.int32),
        pltpu.VMEM((NPAD,), jnp.float32),
        pltpu.SemaphoreType.DMA,
    ],
)()


def _sc_agg_body(hp, src2, dst2, zeros, out, srcg, dstg, rows_v, acc,
                 gsem0, gsem1, ssem0, ssem1, isem):
    c = lax.axis_index("c")
    s = lax.axis_index("s")
    gsems = (gsem0, gsem1)
    ssems = (ssem0, ssem1)
    base = pl.multiple_of(
        jnp.where(c == 0, s * CPW0, NS * CPW0 + s * CPW1), 8)
    cpw = jnp.where(c == 0, CPW0, CPW1)
    ngrp = jnp.where(c == 0, NGRP0, NGRP1)
    pltpu.sync_copy(src2.at[pl.ds(base, GRP)], srcg.at[0])
    pltpu.sync_copy(dst2.at[pl.ds(base, GRP)], dstg.at[0])
    gbase1 = pl.multiple_of(base + GRP, 8)
    pltpu.async_copy(src2.at[pl.ds(gbase1, GRP)], srcg.at[1], isem)
    pltpu.async_copy(dst2.at[pl.ds(gbase1, GRP)], dstg.at[1], isem)
    pltpu.sync_copy(zeros, acc.at[pl.ds(s * RPT, RPT)])
    plsc.subcore_barrier()
    pltpu.async_copy(hp.at[srcg.at[0, 0]], rows_v.at[0], gsem0)

    def body(k, carry):
        g = k // GRP
        p = g % 2
        q = (g + 1) % 2
        r = k % GRP

        for b in range(NBUF):
            @pl.when(k % NBUF == b)
            def _():
                b2 = (b + 1) % NBUF
                pltpu.make_async_copy(hp.at[srcg.at[p, r]], rows_v.at[b],
                                      gsems[b]).wait()
                pltpu.async_copy(rows_v.at[b], acc.at[dstg.at[p, r]],
                                 ssems[b], add=True)

                @pl.when(k > 0)
                def _():
                    pltpu.make_async_copy(
                        rows_v.at[b2], acc.at[dstg.at[p, r]],
                        ssems[b2]).wait()

                @pl.when(jnp.logical_and(jnp.logical_and(r == 0, k > 0),
                                         g + 1 < ngrp))
                def _():
                    gb = pl.multiple_of(base + (g + 1) * GRP, 8)
                    pltpu.async_copy(src2.at[pl.ds(gb, GRP)], srcg.at[q],
                                     isem)
                    pltpu.async_copy(dst2.at[pl.ds(gb, GRP)], dstg.at[q],
                                     isem)

                @pl.when(jnp.logical_and(r == GRP - 1, g + 1 < ngrp))
                def _():
                    pltpu.make_async_copy(src2.at[pl.ds(0, GRP)], srcg.at[q],
                                          isem).wait()
                    pltpu.make_async_copy(dst2.at[pl.ds(0, GRP)], dstg.at[q],
                                          isem).wait()

                @pl.when(k + 1 < cpw)
                def _():
                    g2 = (k + 1) // GRP
                    r2 = (k + 1) % GRP
                    pltpu.async_copy(hp.at[srcg.at[g2 % 2, r2]],
                                     rows_v.at[b2], gsems[b2])
        return carry

    lax.fori_loop(0, cpw, body, 0)
    for b in range(NBUF):
        @pl.when((cpw - 1) % NBUF == b)
        def _():
            pltpu.make_async_copy(rows_v.at[b], acc.at[dstg.at[0, 0]],
                                  ssems[b]).wait()
    plsc.subcore_barrier()
    pltpu.sync_copy(acc.at[pl.ds(s * RPT, RPT)], out.at[c, pl.ds(s * RPT, RPT)])


_sc_agg = functools.partial(
    pl.kernel,
    _sc_agg_body,
    out_type=jax.ShapeDtypeStruct((NC, NPAD, D), jnp.float32),
    mesh=_mesh,
    scratch_types=[
        pltpu.VMEM((2, GRP, CH), jnp.int32),
        pltpu.VMEM((2, GRP, CH), jnp.int32),
        pltpu.VMEM((NBUF, CH, D), jnp.float32),
        pltpu.VMEM_SHARED((NPAD, D), jnp.float32),
        pltpu.SemaphoreType.DMA,
        pltpu.SemaphoreType.DMA,
        pltpu.SemaphoreType.DMA,
        pltpu.SemaphoreType.DMA,
        pltpu.SemaphoreType.DMA,
    ],
)()


def _tc_prep_body(x_ref, w1_ref, cp_ref, z_ref, hp_ref, dinv_ref):
    cnt = jnp.sum(cp_ref[...], axis=0)[:N].reshape(N, 1)
    dinv = lax.rsqrt(cnt + 1.0)
    z = jnp.dot(x_ref[...], w1_ref[...], preferred_element_type=jnp.float32)
    dinv_ref[...] = dinv
    z_ref[...] = z
    hp_ref[...] = z * dinv


def _tc_mid_body(s1_ref, z1_ref, dinv_ref, b1_ref, w2_ref, z2_ref, h2p_ref):
    dinv = dinv_ref[...]
    s = s1_ref[0, :N, :] + s1_ref[1, :N, :]
    z1 = z1_ref[...]
    h1 = jnp.maximum(dinv * s + dinv * dinv * z1 + b1_ref[...], 0.0)
    z2 = jnp.dot(h1, w2_ref[...], preferred_element_type=jnp.float32)
    z2_ref[...] = z2
    h2p_ref[...] = z2 * dinv


def _tc_fin_body(s2_ref, z2_ref, dinv_ref, b2_ref, bat_ref, wfc_ref, bfc_ref,
                 out_ref):
    dinv = dinv_ref[...]
    s = s2_ref[0, :N, :] + s2_ref[1, :N, :]
    z2 = z2_ref[...]
    h2 = jnp.maximum(dinv * s + dinv * dinv * z2 + b2_ref[...], 0.0)
    ids = lax.broadcasted_iota(jnp.int32, (B, N), 0)
    m = (bat_ref[...] == ids).astype(jnp.float32)
    sums = jnp.dot(m, h2, preferred_element_type=jnp.float32)
    cb = jnp.sum(m, axis=1, keepdims=True)
    pooled = sums / jnp.maximum(cb, 1.0)
    logits = jnp.dot(pooled, wfc_ref[...],
                     preferred_element_type=jnp.float32) + bfc_ref[...]
    mx = jnp.max(logits, axis=1, keepdims=True)
    sh = logits - mx
    out_ref[...] = sh - jnp.log(jnp.sum(jnp.exp(sh), axis=1, keepdims=True))


_tc_prep = pl.pallas_call(
    _tc_prep_body,
    out_shape=[
        jax.ShapeDtypeStruct((N, D), jnp.float32),
        jax.ShapeDtypeStruct((N, D), jnp.float32),
        jax.ShapeDtypeStruct((N, 1), jnp.float32),
    ],
)

_tc_mid = pl.pallas_call(
    _tc_mid_body,
    out_shape=[
        jax.ShapeDtypeStruct((N, D), jnp.float32),
        jax.ShapeDtypeStruct((N, D), jnp.float32),
    ],
)

_tc_fin = pl.pallas_call(
    _tc_fin_body,
    out_shape=jax.ShapeDtypeStruct((B, DOUT), jnp.float32),
)


def kernel(x, edge_index, batch, W1, b1, W2, b2, Wfc, bfc):
    src = edge_index[0]
    dst = edge_index[1]
    pad = EPAD - E
    src2 = jnp.concatenate([src, jnp.zeros((pad,), jnp.int32)]).reshape(NROWS, CH)
    dst2 = jnp.concatenate([dst, jnp.full((pad,), N, jnp.int32)]).reshape(NROWS, CH)
    zeros_d = jnp.zeros((RPT, D), jnp.float32)

    cp = _sc_count(dst2)
    z1, h1p, dinv = _tc_prep(x, W1, cp)
    s1 = _sc_agg(h1p, src2, dst2, zeros_d)
    z2, h2p = _tc_mid(s1, z1, dinv, b1.reshape(1, D), W2)
    s2 = _sc_agg(h2p, src2, dst2, zeros_d)
    return _tc_fin(s2, z2, dinv, b2.reshape(1, D), batch.reshape(1, N),
                   Wfc, bfc.reshape(1, DOUT))


# speedup vs baseline: 1.2659x; 1.0016x over previous
import functools

import jax
import jax.numpy as jnp
from jax import lax
from jax.experimental import pallas as pl
from jax.experimental.pallas import tpu as pltpu
from jax.experimental.pallas import tpu_sc as plsc

N = 10000
E = 320000
B = 64
D = 128
DOUT = 64

NC, NS = 2, 16
NW = NC * NS
CH = 128
CPW = 80
EPAD = NW * CPW * CH
NROWS = EPAD // CH
NPAD = 10112
RPT = NPAD // NS

GRP = 8
NBUF = 2
CPW0 = 152
CPW1 = 8
NGRP0 = CPW0 // GRP
NGRP1 = CPW1 // GRP

_mesh = plsc.VectorSubcoreMesh(core_axis_name="c", subcore_axis_name="s")


def _sc_count_body(dst2, out, dst_v, hist, sem):
    c = lax.axis_index("c")
    s = lax.axis_index("s")
    wid = c * NS + s
    pltpu.sync_copy(dst2.at[pl.ds(wid * CPW, CPW)], dst_v)
    zeros16 = jnp.zeros((16,), jnp.float32)
    ones16 = jnp.ones((16,), jnp.float32)

    def zbody(i, carry):
        hist[pl.ds(i * 16, 16)] = zeros16
        return carry

    lax.fori_loop(0, NPAD // 16, zbody, 0)

    def body(i, carry):
        idx = dst_v[i // (CH // 16), pl.ds((i % (CH // 16)) * 16, 16)]
        plsc.addupdate_scatter(hist, [idx], ones16)
        return carry

    lax.fori_loop(0, (CPW * CH) // 16, body, 0)
    pltpu.sync_copy(hist, out.at[wid])


_sc_count = functools.partial(
    pl.kernel,
    _sc_count_body,
    out_type=jax.ShapeDtypeStruct((NW, NPAD), jnp.float32),
    mesh=_mesh,
    compiler_params=pltpu.CompilerParams(needs_layout_passes=False),
    scratch_types=[
        pltpu.VMEM((CPW, CH), jnp.int32),
        pltpu.VMEM((NPAD,), jnp.float32),
        pltpu.SemaphoreType.DMA,
    ],
)()


def _sc_agg_body(hp, src2, dst2, zeros, out, srcg, dstg, rows_v, acc,
                 gsem0, gsem1, ssem0, ssem1, isem):
    c = lax.axis_index("c")
    s = lax.axis_index("s")
    gsems = (gsem0, gsem1)
    ssems = (ssem0, ssem1)
    base = pl.multiple_of(
        jnp.where(c == 0, s * CPW0, NS * CPW0 + s * CPW1), 8)
    cpw = jnp.where(c == 0, CPW0, CPW1)
    ngrp = jnp.where(c == 0, NGRP0, NGRP1)
    pltpu.sync_copy(src2.at[pl.ds(base, GRP)], srcg.at[0])
    pltpu.sync_copy(dst2.at[pl.ds(base, GRP)], dstg.at[0])

    @pl.when(ngrp > 1)
    def _():
        gbase1 = pl.multiple_of(base + GRP, 8)
        pltpu.async_copy(src2.at[pl.ds(gbase1, GRP)], srcg.at[1], isem)
        pltpu.async_copy(dst2.at[pl.ds(gbase1, GRP)], dstg.at[1], isem)
    pltpu.sync_copy(zeros, acc.at[pl.ds(s * RPT, RPT)])
    plsc.subcore_barrier()
    pltpu.async_copy(hp.at[srcg.at[0, 0]], rows_v.at[0], gsem0)

    def body(k, carry):
        g = k // GRP
        p = g % 2
        q = (g + 1) % 2
        r = k % GRP

        for b in range(NBUF):
            @pl.when(k % NBUF == b)
            def _():
                b2 = (b + 1) % NBUF
                pltpu.make_async_copy(hp.at[srcg.at[p, r]], rows_v.at[b],
                                      gsems[b]).wait()
                pltpu.async_copy(rows_v.at[b], acc.at[dstg.at[p, r]],
                                 ssems[b], add=True)

                @pl.when(k > 0)
                def _():
                    pltpu.make_async_copy(
                        rows_v.at[b2], acc.at[dstg.at[p, r]],
                        ssems[b2]).wait()

                @pl.when(jnp.logical_and(jnp.logical_and(r == 0, k > 0),
                                         g + 1 < ngrp))
                def _():
                    gb = pl.multiple_of(base + (g + 1) * GRP, 8)
                    pltpu.async_copy(src2.at[pl.ds(gb, GRP)], srcg.at[q],
                                     isem)
                    pltpu.async_copy(dst2.at[pl.ds(gb, GRP)], dstg.at[q],
                                     isem)

                @pl.when(jnp.logical_and(r == GRP - 1, g + 1 < ngrp))
                def _():
                    pltpu.make_async_copy(src2.at[pl.ds(0, GRP)], srcg.at[q],
                                          isem).wait()
                    pltpu.make_async_copy(dst2.at[pl.ds(0, GRP)], dstg.at[q],
                                          isem).wait()

                @pl.when(k + 1 < cpw)
                def _():
                    g2 = (k + 1) // GRP
                    r2 = (k + 1) % GRP
                    pltpu.async_copy(hp.at[srcg.at[g2 % 2, r2]],
                                     rows_v.at[b2], gsems[b2])
        return carry

    lax.fori_loop(0, cpw, body, 0)
    for b in range(NBUF):
        @pl.when((cpw - 1) % NBUF == b)
        def _():
            pltpu.make_async_copy(rows_v.at[b], acc.at[dstg.at[0, 0]],
                                  ssems[b]).wait()
    plsc.subcore_barrier()
    pltpu.sync_copy(acc.at[pl.ds(s * RPT, RPT)], out.at[c, pl.ds(s * RPT, RPT)])


_sc_agg = functools.partial(
    pl.kernel,
    _sc_agg_body,
    out_type=jax.ShapeDtypeStruct((NC, NPAD, D), jnp.float32),
    mesh=_mesh,
    scratch_types=[
        pltpu.VMEM((2, GRP, CH), jnp.int32),
        pltpu.VMEM((2, GRP, CH), jnp.int32),
        pltpu.VMEM((NBUF, CH, D), jnp.float32),
        pltpu.VMEM_SHARED((NPAD, D), jnp.float32),
        pltpu.SemaphoreType.DMA,
        pltpu.SemaphoreType.DMA,
        pltpu.SemaphoreType.DMA,
        pltpu.SemaphoreType.DMA,
        pltpu.SemaphoreType.DMA,
    ],
)()


def _tc_prep_body(x_ref, w1_ref, cp_ref, z_ref, hp_ref, dinv_ref):
    cnt = jnp.sum(cp_ref[...], axis=0)[:N].reshape(N, 1)
    dinv = lax.rsqrt(cnt + 1.0)
    z = jnp.dot(x_ref[...], w1_ref[...], preferred_element_type=jnp.float32)
    dinv_ref[...] = dinv
    z_ref[...] = z
    hp_ref[...] = z * dinv


def _tc_mid_body(s1_ref, z1_ref, dinv_ref, b1_ref, w2_ref, z2_ref, h2p_ref):
    dinv = dinv_ref[...]
    s = s1_ref[0, :N, :] + s1_ref[1, :N, :]
    z1 = z1_ref[...]
    h1 = jnp.maximum(dinv * s + dinv * dinv * z1 + b1_ref[...], 0.0)
    z2 = jnp.dot(h1, w2_ref[...], preferred_element_type=jnp.float32)
    z2_ref[...] = z2
    h2p_ref[...] = z2 * dinv


def _tc_fin_body(s2_ref, z2_ref, dinv_ref, b2_ref, bat_ref, wfc_ref, bfc_ref,
                 out_ref):
    dinv = dinv_ref[...]
    s = s2_ref[0, :N, :] + s2_ref[1, :N, :]
    z2 = z2_ref[...]
    h2 = jnp.maximum(dinv * s + dinv * dinv * z2 + b2_ref[...], 0.0)
    ids = lax.broadcasted_iota(jnp.int32, (B, N), 0)
    m = (bat_ref[...] == ids).astype(jnp.float32)
    sums = jnp.dot(m, h2, preferred_element_type=jnp.float32)
    cb = jnp.sum(m, axis=1, keepdims=True)
    pooled = sums / jnp.maximum(cb, 1.0)
    logits = jnp.dot(pooled, wfc_ref[...],
                     preferred_element_type=jnp.float32) + bfc_ref[...]
    mx = jnp.max(logits, axis=1, keepdims=True)
    sh = logits - mx
    out_ref[...] = sh - jnp.log(jnp.sum(jnp.exp(sh), axis=1, keepdims=True))


_tc_prep = pl.pallas_call(
    _tc_prep_body,
    out_shape=[
        jax.ShapeDtypeStruct((N, D), jnp.float32),
        jax.ShapeDtypeStruct((N, D), jnp.float32),
        jax.ShapeDtypeStruct((N, 1), jnp.float32),
    ],
)

_tc_mid = pl.pallas_call(
    _tc_mid_body,
    out_shape=[
        jax.ShapeDtypeStruct((N, D), jnp.float32),
        jax.ShapeDtypeStruct((N, D), jnp.float32),
    ],
)

_tc_fin = pl.pallas_call(
    _tc_fin_body,
    out_shape=jax.ShapeDtypeStruct((B, DOUT), jnp.float32),
)


def kernel(x, edge_index, batch, W1, b1, W2, b2, Wfc, bfc):
    src = edge_index[0]
    dst = edge_index[1]
    pad = EPAD - E
    src2 = jnp.concatenate([src, jnp.zeros((pad,), jnp.int32)]).reshape(NROWS, CH)
    dst2 = jnp.concatenate([dst, jnp.full((pad,), N, jnp.int32)]).reshape(NROWS, CH)
    zeros_d = jnp.zeros((RPT, D), jnp.float32)

    cp = _sc_count(dst2)
    z1, h1p, dinv = _tc_prep(x, W1, cp)
    s1 = _sc_agg(h1p, src2, dst2, zeros_d)
    z2, h2p = _tc_mid(s1, z1, dinv, b1.reshape(1, D), W2)
    s2 = _sc_agg(h2p, src2, dst2, zeros_d)
    return _tc_fin(s2, z2, dinv, b2.reshape(1, D), batch.reshape(1, N),
                   Wfc, bfc.reshape(1, DOUT))
